# baseline (speedup 1.0000x reference)
import functools

import jax
import jax.numpy as jnp
from jax import lax
from jax.experimental import pallas as pl
from jax.experimental.pallas import tpu as pltpu
from jax.experimental.pallas import tpu_sc as plsc

NN = 10000
NE = 320000
DF = 128
DH = 16
DC = 40

NCORES = 2
NSUB = 16
NW = NCORES * NSUB
EPT = NE // NW
EPC = NE // NSUB
CH = 80
NCHUNK = EPT // CH
NBUF = 5
DROW = NN // 16
DPAD = 640
NR = NN // NSUB
NB = 624
NBL = NN - 15 * NB

_mesh = plsc.VectorSubcoreMesh(core_axis_name="c", subcore_axis_name="s")
_sc_params = pltpu.CompilerParams(needs_layout_passes=False,
                                  use_tc_tiling_on_sc=False)


def _newton_rsqrt(x):
    i = plsc.bitcast(x, jnp.int32)
    i = jnp.int32(0x5F3759DF) - (i >> 1)
    y = plsc.bitcast(i, jnp.float32)
    xh = x * 0.5
    for _ in range(4):
        y = y * (1.5 - xh * y * y)
    return y


def _zero_acc(msg_v, acc_sh, base):
    for j in range(7):
        pltpu.sync_copy(msg_v.at[0], acc_sh.at[pl.ds(base + j * CH, CH)])
    pltpu.sync_copy(msg_v.at[1].at[pl.ds(0, NR - 7 * CH)],
                    acc_sh.at[pl.ds(base + 7 * CH, NR - 7 * CH)])


def _edge_pass(h_src, acc_sh, row_v, col2_v, cw_v, msg_v, gsems, ssems):
    def start_gather(c, b):
        pltpu.async_copy(h_src.at[row_v.at[pl.ds(c * CH, CH)]],
                         msg_v.at[b], gsems[b])

    def wait_gather(c, b):
        pltpu.make_async_copy(h_src.at[row_v.at[pl.ds(c * CH, CH)]],
                              msg_v.at[b], gsems[b]).wait()

    def start_scatter(c, b):
        pltpu.async_copy(msg_v.at[b], acc_sh.at[col2_v.at[c]], ssems[b],
                         add=True)

    def wait_scatter(c, b):
        pltpu.make_async_copy(msg_v.at[b], acc_sh.at[col2_v.at[c]],
                              ssems[b]).wait()

    for b in range(3):
        start_gather(b, b)

    def group(g, carry):
        for b in range(NBUF):
            c = g * NBUF + b
            bn = (b + 3) % NBUF
            if b < 2:
                @pl.when(g >= 1)
                def _():
                    wait_scatter(c - 2, bn)
                start_gather(c + 3, bn)
            else:
                @pl.when(g < (NCHUNK // NBUF) - 1)
                def _():
                    wait_scatter(c - 2, bn)
                    start_gather(c + 3, bn)
            wait_gather(c, b)
            for gg in range(CH // 16):
                wvec = cw_v[pl.ds(c * CH + gg * 16, 16)]
                for e in range(16):
                    r = gg * 16 + e
                    msg_v[b, r, :] = msg_v[b, r, :] * wvec[e]
            start_scatter(c, b)
        return carry

    lax.fori_loop(0, NCHUNK // NBUF, group, 0)
    for c in range(NCHUNK - NBUF, NCHUNK):
        wait_scatter(c, c % NBUF)


PMASK = (1 << 14) - 1


@functools.partial(
    pl.kernel,
    out_type=[
        jax.ShapeDtypeStruct((NCORES, NN, DH), jnp.float32),
        jax.ShapeDtypeStruct((NE,), jnp.float32),
        jax.ShapeDtypeStruct((NN,), jnp.float32),
    ],
    mesh=_mesh,
    compiler_params=_sc_params,
    scratch_types=[
        pltpu.VMEM((EPC,), jnp.int32),
        pltpu.VMEM((EPC,), jnp.float32),
        pltpu.VMEM((DROW, 16), jnp.float32),
        pltpu.VMEM((5, 125), jnp.int32),
        pltpu.VMEM((NN,), jnp.float32),
        pltpu.VMEM((NN,), jnp.float32),
        pltpu.VMEM((EPT,), jnp.int32),
        pltpu.VMEM((NCHUNK, CH), jnp.int32),
        pltpu.VMEM((EPT,), jnp.float32),
        pltpu.VMEM((NBUF, CH, DH), jnp.float32),
        pltpu.VMEM_SHARED((DPAD, 16), jnp.float32),
        pltpu.VMEM_SHARED((NN, DH), jnp.float32),
        [pltpu.SemaphoreType.DMA] * NBUF,
        [pltpu.SemaphoreType.DMA] * NBUF,
    ],
)
def _sc_k1(h1_hbm, pc_hbm, ea_hbm, p1_hbm, cw_hbm, ds2_hbm,
           dpc_v, dew_v, dacc_v, idx2_v, disp_v, ds2_v,
           row_v, col2_v, cw_v, msg_v,
           deg_sh, acc_sh, gsems, ssems):
    cid = lax.axis_index("c")
    sid = lax.axis_index("s")
    wid = sid * NCORES + cid
    eoff = cid * EPT

    pltpu.sync_copy(pc_hbm.at[pl.ds(sid * EPC, EPC)], dpc_v)
    pltpu.sync_copy(ea_hbm.at[pl.ds(sid * EPC, EPC)], dew_v)

    zeros = jnp.zeros((16,), jnp.float32)

    def zmsg(i, carry):
        msg_v[i // CH, i % CH, :] = zeros
        return carry

    lax.fori_loop(0, NBUF * CH, zmsg, 0, unroll=8)

    def zacc(r, carry):
        dacc_v[r, :] = zeros
        return carry

    lax.fori_loop(0, DROW, zacc, 0, unroll=8)

    iota = lax.iota(jnp.int32, 16)
    for j in range(5):
        for i in range(8):
            base = min(i * 16, 125 - 16)
            idx2_v[j, pl.ds(base, 16)] = iota + (j * 125 + base)

    _zero_acc(msg_v, acc_sh, sid * NR)
    pltpu.sync_copy(msg_v.at[2].at[pl.ds(0, DPAD // NSUB)],
                    deg_sh.at[pl.ds(sid * (DPAD // NSUB), DPAD // NSUB)])
    plsc.subcore_barrier()

    def dbody(i, carry):
        c16 = dpc_v[pl.ds(i * 16, 16)] & PMASK
        w16 = dew_v[pl.ds(i * 16, 16)]
        plsc.addupdate_scatter(dacc_v, [c16 >> 4, c16 & 15], w16)
        return carry

    lax.fori_loop(0, EPC // 16, dbody, 0, unroll=4)
    for j in range(5):
        pltpu.sync_copy(dacc_v.at[pl.ds(j * 125, 125)],
                        deg_sh.at[idx2_v.at[j]], add=True)
    plsc.subcore_barrier()

    pltpu.sync_copy(deg_sh.at[pl.ds(0, DROW)], dacc_v)

    def newton(r, carry):
        y = _newton_rsqrt(dacc_v[r, :] + 1.0)
        disp_v[pl.ds(r * 16, 16)] = y
        ds2_v[pl.ds(r * 16, 16)] = y * y
        return carry

    lax.fori_loop(0, DROW, newton, 0, unroll=4)

    def cwbody(i, carry):
        s = pl.ds(i * 16, 16)
        p16 = dpc_v[pl.ds(eoff + i * 16, 16)]
        r16 = p16 >> 14
        c16 = p16 & PMASK
        row_v[s] = r16
        dr = plsc.load_gather(disp_v, [r16])
        dc = plsc.load_gather(disp_v, [c16])
        cw_v[s] = dew_v[pl.ds(eoff + i * 16, 16)] * dr * dc
        return carry

    lax.fori_loop(0, EPT // 16, cwbody, 0, unroll=4)
    pltpu.sync_copy(cw_v, cw_hbm.at[pl.ds(wid * EPT, EPT)])

    @pl.when(jnp.logical_and(cid == 0, sid < 15))
    def _():
        pltpu.sync_copy(ds2_v.at[pl.ds(sid * NB, NB)],
                        ds2_hbm.at[pl.ds(sid * NB, NB)])

    @pl.when(jnp.logical_and(cid == 0, sid == 15))
    def _():
        pltpu.sync_copy(ds2_v.at[pl.ds(15 * NB, NBL)],
                        ds2_hbm.at[pl.ds(15 * NB, NBL)])

    def repack(i, carry):
        c = i // (CH // 16)
        g = i % (CH // 16)
        col2_v[c, pl.ds(g * 16, 16)] = dpc_v[pl.ds(eoff + i * 16, 16)] & PMASK
        return carry

    lax.fori_loop(0, NCHUNK * (CH // 16), repack, 0, unroll=8)

    _edge_pass(h1_hbm, acc_sh, row_v, col2_v, cw_v, msg_v, gsems, ssems)
    plsc.subcore_barrier()
    pltpu.sync_copy(acc_sh.at[pl.ds(sid * NR, NR)],
                    p1_hbm.at[cid, pl.ds(sid * NR, NR)])


@functools.partial(
    pl.kernel,
    out_type=[
        jax.ShapeDtypeStruct((NCORES, NN, DH), jnp.float32),
        jax.ShapeDtypeStruct((NN, DH), jnp.float32),
    ],
    mesh=_mesh,
    compiler_params=_sc_params,
    scratch_types=[
        pltpu.VMEM((EPT,), jnp.int32),
        pltpu.VMEM((EPT,), jnp.int32),
        pltpu.VMEM((NCHUNK, CH), jnp.int32),
        pltpu.VMEM((EPT,), jnp.float32),
        pltpu.VMEM((NBUF, CH, DH), jnp.float32),
        pltpu.VMEM((NBL, DH), jnp.float32),
        pltpu.VMEM((NBL, DH), jnp.float32),
        pltpu.VMEM((NBL, DH), jnp.float32),
        pltpu.VMEM((NBL,), jnp.float32),
        pltpu.VMEM((NBL, DH), jnp.float32),
        pltpu.VMEM((NBL, DH), jnp.float32),
        pltpu.VMEM((1, DH), jnp.float32),
        pltpu.VMEM_SHARED((NN, DH), jnp.float32),
        pltpu.VMEM_SHARED((NN, DH), jnp.float32),
        [pltpu.SemaphoreType.DMA] * NBUF,
        [pltpu.SemaphoreType.DMA] * NBUF,
    ],
)
def _sc_k2(p1_hbm, h1_hbm, ds2_hbm, b1_hbm, pc_hbm, cw_hbm,
           p2_hbm, u_hbm,
           pc_v, row_v, col2_v, cw_v, msg_v,
           p1a_v, p1b_v, h1_v, ds2_v, z1_v, u_v, b1_v,
           z1_sh, acc_sh, gsems, ssems):
    cid = lax.axis_index("c")
    sid = lax.axis_index("s")
    wid = sid * NCORES + cid

    pltpu.sync_copy(pc_hbm.at[pl.ds(wid * EPT, EPT)], pc_v)
    pltpu.sync_copy(cw_hbm.at[pl.ds(wid * EPT, EPT)], cw_v)
    pltpu.sync_copy(b1_hbm, b1_v)

    @pl.when(sid < 15)
    def _():
        nr = pl.ds(sid * NB, NB)
        vr = pl.ds(0, NB)
        pltpu.sync_copy(p1_hbm.at[0, nr], p1a_v.at[vr])
        pltpu.sync_copy(p1_hbm.at[1, nr], p1b_v.at[vr])
        pltpu.sync_copy(h1_hbm.at[nr], h1_v.at[vr])
        pltpu.sync_copy(ds2_hbm.at[nr], ds2_v.at[vr])

    @pl.when(sid == 15)
    def _():
        nr = pl.ds(15 * NB, NBL)
        pltpu.sync_copy(p1_hbm.at[0, nr], p1a_v)
        pltpu.sync_copy(p1_hbm.at[1, nr], p1b_v)
        pltpu.sync_copy(h1_hbm.at[nr], h1_v)
        pltpu.sync_copy(ds2_hbm.at[nr], ds2_v)

    zeros = jnp.zeros((16,), jnp.float32)

    def zmsg(i, carry):
        msg_v[i // CH, i % CH, :] = zeros
        return carry

    lax.fori_loop(0, NBUF * CH, zmsg, 0, unroll=8)
    _zero_acc(msg_v, acc_sh, sid * NR)

    b1vec = b1_v[0, :]

    def mid(r, carry):
        d2 = plsc.load_gather(ds2_v, [jnp.full((16,), r, jnp.int32)])
        z = p1a_v[r, :] + p1b_v[r, :] + d2 * h1_v[r, :] + b1vec
        z1 = jnp.maximum(z, 0.0)
        z1_v[r, :] = z1
        u_v[r, :] = d2 * z1
        return carry

    lax.fori_loop(0, NBL, mid, 0, unroll=4)

    @pl.when(sid < 15)
    def _():
        nr = pl.ds(sid * NB, NB)
        vr = pl.ds(0, NB)
        pltpu.sync_copy(z1_v.at[vr], z1_sh.at[nr])

    @pl.when(sid == 15)
    def _():
        pltpu.sync_copy(z1_v, z1_sh.at[pl.ds(15 * NB, NBL)])

    @pl.when(jnp.logical_and(cid == 0, sid < 15))
    def _():
        nr = pl.ds(sid * NB, NB)
        pltpu.sync_copy(u_v.at[pl.ds(0, NB)], u_hbm.at[nr])

    @pl.when(jnp.logical_and(cid == 0, sid == 15))
    def _():
        pltpu.sync_copy(u_v, u_hbm.at[pl.ds(15 * NB, NBL)])

    def unpack(i, carry):
        c = i // (CH // 16)
        g = i % (CH // 16)
        p16 = pc_v[pl.ds(i * 16, 16)]
        row_v[pl.ds(i * 16, 16)] = p16 >> 14
        col2_v[c, pl.ds(g * 16, 16)] = p16 & PMASK
        return carry

    lax.fori_loop(0, NCHUNK * (CH // 16), unpack, 0, unroll=8)
    plsc.subcore_barrier()

    _edge_pass(z1_sh, acc_sh, row_v, col2_v, cw_v, msg_v,
               gsems, ssems)
    plsc.subcore_barrier()
    pltpu.sync_copy(acc_sh.at[pl.ds(sid * NR, NR)],
                    p2_hbm.at[cid, pl.ds(sid * NR, NR)])


_BR = 2000


def _tc_mm_body(x_ref, w1_ref, h1_ref):
    h1_ref[...] = jnp.dot(x_ref[...], w1_ref[...],
                          preferred_element_type=jnp.float32)


def _tc_mm(x, w1):
    return pl.pallas_call(
        _tc_mm_body,
        grid=(NN // _BR,),
        in_specs=[
            pl.BlockSpec((_BR, DF), lambda i: (i, 0)),
            pl.BlockSpec((DF, DH), lambda i: (0, 0)),
        ],
        out_specs=pl.BlockSpec((_BR, DH), lambda i: (i, 0)),
        out_shape=jax.ShapeDtypeStruct((NN, DH), jnp.float32),
    )(x, w1)


def _tc_post_body(p_ref, u_ref, w2_ref, b2_ref, out_ref):
    agg = p_ref[0] + p_ref[1] + u_ref[...]
    o = jnp.dot(agg, w2_ref[...], preferred_element_type=jnp.float32)
    o = o + b2_ref[...]
    m = jnp.max(o, axis=1, keepdims=True)
    lse = jnp.log(jnp.sum(jnp.exp(o - m), axis=1, keepdims=True)) + m
    out_ref[...] = o - lse


def _tc_post(p, u, w2, b2):
    return pl.pallas_call(
        _tc_post_body,
        grid=(NN // _BR,),
        in_specs=[
            pl.BlockSpec((NCORES, _BR, DH), lambda i: (0, i, 0)),
            pl.BlockSpec((_BR, DH), lambda i: (i, 0)),
            pl.BlockSpec((DH, DC), lambda i: (0, 0)),
            pl.BlockSpec((1, DC), lambda i: (0, 0)),
        ],
        out_specs=pl.BlockSpec((_BR, DC), lambda i: (i, 0)),
        out_shape=jax.ShapeDtypeStruct((NN, DC), jnp.float32),
    )(p, u, w2, b2)


def kernel(x, edge_index, edge_attr, W1, b1, W2, b2):
    ei = edge_index.astype(jnp.int32)
    pc = (ei[0] << 14) | ei[1]
    ea = edge_attr.astype(jnp.float32)

    h1 = _tc_mm(x, W1)
    p1, cw, ds2 = _sc_k1(h1, pc, ea)
    p2, u = _sc_k2(p1, h1, ds2, b1.reshape(1, DH), pc, cw)
    return _tc_post(p2, u, W2, b2.reshape(1, DC))


# speedup vs baseline: 1.0856x; 1.0856x over previous
import functools

import jax
import jax.numpy as jnp
from jax import lax
from jax.experimental import pallas as pl
from jax.experimental.pallas import tpu as pltpu
from jax.experimental.pallas import tpu_sc as plsc

NN = 10000
NE = 320000
DF = 128
DH = 16
DC = 40

NCORES = 2
NSUB = 16
NW = NCORES * NSUB
EPT = NE // NW
EPC = NE // NSUB
CH = 80
NCHUNK = EPT // CH
NBUF = 5
DROW = NN // 16
DPAD = 640
NR = NN // NSUB
NB = 624
NBL = NN - 15 * NB

_mesh = plsc.VectorSubcoreMesh(core_axis_name="c", subcore_axis_name="s")
_sc_params = pltpu.CompilerParams(needs_layout_passes=False,
                                  use_tc_tiling_on_sc=False)


def _newton_rsqrt(x):
    i = plsc.bitcast(x, jnp.int32)
    i = jnp.int32(0x5F3759DF) - (i >> 1)
    y = plsc.bitcast(i, jnp.float32)
    xh = x * 0.5
    for _ in range(4):
        y = y * (1.5 - xh * y * y)
    return y


def _zero_acc(msg_v, acc_sh, base):
    for j in range(7):
        pltpu.sync_copy(msg_v.at[0], acc_sh.at[pl.ds(base + j * CH, CH)])
    pltpu.sync_copy(msg_v.at[1].at[pl.ds(0, NR - 7 * CH)],
                    acc_sh.at[pl.ds(base + 7 * CH, NR - 7 * CH)])


def _edge_pass(h_src, acc_sh, row_v, col2_v, cw_v, msg_v, gsems, ssems):
    def start_gather(c, b):
        pltpu.async_copy(h_src.at[row_v.at[pl.ds(c * CH, CH)]],
                         msg_v.at[b], gsems[b])

    def wait_gather(c, b):
        pltpu.make_async_copy(h_src.at[row_v.at[pl.ds(c * CH, CH)]],
                              msg_v.at[b], gsems[b]).wait()

    def start_scatter(c, b):
        pltpu.async_copy(msg_v.at[b], acc_sh.at[col2_v.at[c]], ssems[b],
                         add=True)

    def wait_scatter(c, b):
        pltpu.make_async_copy(msg_v.at[b], acc_sh.at[col2_v.at[c]],
                              ssems[b]).wait()

    for b in range(3):
        start_gather(b, b)

    def group(g, carry):
        for b in range(NBUF):
            c = g * NBUF + b
            bn = (b + 3) % NBUF
            if b < 2:
                @pl.when(g >= 1)
                def _():
                    wait_scatter(c - 2, bn)
                start_gather(c + 3, bn)
            else:
                @pl.when(g < (NCHUNK // NBUF) - 1)
                def _():
                    wait_scatter(c - 2, bn)
                    start_gather(c + 3, bn)
            wait_gather(c, b)
            for gg in range(CH // 16):
                wvec = cw_v[pl.ds(c * CH + gg * 16, 16)]
                for e in range(16):
                    r = gg * 16 + e
                    msg_v[b, r, :] = msg_v[b, r, :] * wvec[e]
            start_scatter(c, b)
        return carry

    lax.fori_loop(0, NCHUNK // NBUF, group, 0)
    for c in range(NCHUNK - NBUF, NCHUNK):
        wait_scatter(c, c % NBUF)


PMASK = (1 << 14) - 1


@functools.partial(
    pl.kernel,
    out_type=[
        jax.ShapeDtypeStruct((NCORES, NN, DH), jnp.float32),
        jax.ShapeDtypeStruct((NE,), jnp.float32),
        jax.ShapeDtypeStruct((NN,), jnp.float32),
    ],
    mesh=_mesh,
    compiler_params=_sc_params,
    scratch_types=[
        pltpu.VMEM((EPC,), jnp.int32),
        pltpu.VMEM((EPC,), jnp.float32),
        pltpu.VMEM((DROW, 16), jnp.float32),
        pltpu.VMEM((5, 125), jnp.int32),
        pltpu.VMEM((NN,), jnp.float32),
        pltpu.VMEM((NN,), jnp.float32),
        pltpu.VMEM((EPT,), jnp.int32),
        pltpu.VMEM((NCHUNK, CH), jnp.int32),
        pltpu.VMEM((EPT,), jnp.float32),
        pltpu.VMEM((NBUF, CH, DH), jnp.float32),
        pltpu.VMEM_SHARED((DPAD, 16), jnp.float32),
        pltpu.VMEM_SHARED((NN, DH), jnp.float32),
        [pltpu.SemaphoreType.DMA] * NBUF,
        [pltpu.SemaphoreType.DMA] * NBUF,
    ],
)
def _sc_k1(h1_hbm, pc_hbm, ea_hbm, p1_hbm, cw_hbm, ds2_hbm,
           dpc_v, dew_v, dacc_v, idx2_v, disp_v, ds2_v,
           row_v, col2_v, cw_v, msg_v,
           deg_sh, acc_sh, gsems, ssems):
    cid = lax.axis_index("c")
    sid = lax.axis_index("s")
    wid = sid * NCORES + cid
    eoff = cid * EPT

    pltpu.sync_copy(pc_hbm.at[pl.ds(sid * EPC, EPC)], dpc_v)
    pltpu.sync_copy(ea_hbm.at[pl.ds(sid * EPC, EPC)], dew_v)

    zeros = jnp.zeros((16,), jnp.float32)

    def zmsg(i, carry):
        msg_v[i // CH, i % CH, :] = zeros
        return carry

    lax.fori_loop(0, NBUF * CH, zmsg, 0, unroll=8)

    def zacc(r, carry):
        dacc_v[r, :] = zeros
        return carry

    lax.fori_loop(0, DROW, zacc, 0, unroll=8)

    iota = lax.iota(jnp.int32, 16)
    for j in range(5):
        for i in range(8):
            base = min(i * 16, 125 - 16)
            idx2_v[j, pl.ds(base, 16)] = iota + (j * 125 + base)

    _zero_acc(msg_v, acc_sh, sid * NR)
    pltpu.sync_copy(msg_v.at[2].at[pl.ds(0, DPAD // NSUB)],
                    deg_sh.at[pl.ds(sid * (DPAD // NSUB), DPAD // NSUB)])
    plsc.subcore_barrier()

    def dbody(i, carry):
        c16 = dpc_v[pl.ds(i * 16, 16)] & PMASK
        w16 = dew_v[pl.ds(i * 16, 16)]
        plsc.addupdate_scatter(dacc_v, [c16 >> 4, c16 & 15], w16)
        return carry

    lax.fori_loop(0, EPC // 16, dbody, 0, unroll=4)
    for j in range(5):
        pltpu.sync_copy(dacc_v.at[pl.ds(j * 125, 125)],
                        deg_sh.at[idx2_v.at[j]], add=True)
    plsc.subcore_barrier()

    pltpu.sync_copy(deg_sh.at[pl.ds(0, DROW)], dacc_v)

    def newton(r, carry):
        y = _newton_rsqrt(dacc_v[r, :] + 1.0)
        disp_v[pl.ds(r * 16, 16)] = y
        ds2_v[pl.ds(r * 16, 16)] = y * y
        return carry

    lax.fori_loop(0, DROW, newton, 0, unroll=4)

    def cwbody(i, carry):
        s = pl.ds(i * 16, 16)
        p16 = dpc_v[pl.ds(eoff + i * 16, 16)]
        r16 = p16 >> 14
        c16 = p16 & PMASK
        row_v[s] = r16
        dr = plsc.load_gather(disp_v, [r16])
        dc = plsc.load_gather(disp_v, [c16])
        cw_v[s] = dew_v[pl.ds(eoff + i * 16, 16)] * dr * dc
        return carry

    lax.fori_loop(0, EPT // 16, cwbody, 0, unroll=4)
    pltpu.sync_copy(cw_v, cw_hbm.at[pl.ds(wid * EPT, EPT)])

    @pl.when(jnp.logical_and(cid == 0, sid < 15))
    def _():
        pltpu.sync_copy(ds2_v.at[pl.ds(sid * NB, NB)],
                        ds2_hbm.at[pl.ds(sid * NB, NB)])

    @pl.when(jnp.logical_and(cid == 0, sid == 15))
    def _():
        pltpu.sync_copy(ds2_v.at[pl.ds(15 * NB, NBL)],
                        ds2_hbm.at[pl.ds(15 * NB, NBL)])

    def repack(i, carry):
        c = i // (CH // 16)
        g = i % (CH // 16)
        col2_v[c, pl.ds(g * 16, 16)] = dpc_v[pl.ds(eoff + i * 16, 16)] & PMASK
        return carry

    lax.fori_loop(0, NCHUNK * (CH // 16), repack, 0, unroll=8)

    _edge_pass(h1_hbm, acc_sh, row_v, col2_v, cw_v, msg_v, gsems, ssems)
    plsc.subcore_barrier()
    pltpu.sync_copy(acc_sh.at[pl.ds(sid * NR, NR)],
                    p1_hbm.at[cid, pl.ds(sid * NR, NR)])


@functools.partial(
    pl.kernel,
    out_type=[
        jax.ShapeDtypeStruct((NCORES, NN, DH), jnp.float32),
        jax.ShapeDtypeStruct((NN, DH), jnp.float32),
    ],
    mesh=_mesh,
    compiler_params=_sc_params,
    scratch_types=[
        pltpu.VMEM((EPT,), jnp.int32),
        pltpu.VMEM((EPT,), jnp.int32),
        pltpu.VMEM((NCHUNK, CH), jnp.int32),
        pltpu.VMEM((EPT,), jnp.float32),
        pltpu.VMEM((NBUF, CH, DH), jnp.float32),
        pltpu.VMEM((NBL, DH), jnp.float32),
        pltpu.VMEM((NBL, DH), jnp.float32),
        pltpu.VMEM((NBL, DH), jnp.float32),
        pltpu.VMEM((NBL,), jnp.float32),
        pltpu.VMEM((NBL, DH), jnp.float32),
        pltpu.VMEM((NBL, DH), jnp.float32),
        pltpu.VMEM((1, DH), jnp.float32),
        pltpu.VMEM_SHARED((NN, DH), jnp.float32),
        pltpu.VMEM_SHARED((NN, DH), jnp.float32),
        [pltpu.SemaphoreType.DMA] * NBUF,
        [pltpu.SemaphoreType.DMA] * NBUF,
    ],
)
def _sc_k2(p1_hbm, h1_hbm, ds2_hbm, b1_hbm, pc_hbm, cw_hbm,
           p2_hbm, u_hbm,
           pc_v, row_v, col2_v, cw_v, msg_v,
           p1a_v, p1b_v, h1_v, ds2_v, z1_v, u_v, b1_v,
           z1_sh, acc_sh, gsems, ssems):
    cid = lax.axis_index("c")
    sid = lax.axis_index("s")
    wid = sid * NCORES + cid

    pltpu.sync_copy(pc_hbm.at[pl.ds(wid * EPT, EPT)], pc_v)
    pltpu.sync_copy(cw_hbm.at[pl.ds(wid * EPT, EPT)], cw_v)
    pltpu.sync_copy(b1_hbm, b1_v)

    @pl.when(sid < 15)
    def _():
        nr = pl.ds(sid * NB, NB)
        vr = pl.ds(0, NB)
        pltpu.sync_copy(p1_hbm.at[0, nr], p1a_v.at[vr])
        pltpu.sync_copy(p1_hbm.at[1, nr], p1b_v.at[vr])
        pltpu.sync_copy(h1_hbm.at[nr], h1_v.at[vr])
        pltpu.sync_copy(ds2_hbm.at[nr], ds2_v.at[vr])

    @pl.when(sid == 15)
    def _():
        nr = pl.ds(15 * NB, NBL)
        pltpu.sync_copy(p1_hbm.at[0, nr], p1a_v)
        pltpu.sync_copy(p1_hbm.at[1, nr], p1b_v)
        pltpu.sync_copy(h1_hbm.at[nr], h1_v)
        pltpu.sync_copy(ds2_hbm.at[nr], ds2_v)

    zeros = jnp.zeros((16,), jnp.float32)

    def zmsg(i, carry):
        msg_v[i // CH, i % CH, :] = zeros
        return carry

    lax.fori_loop(0, NBUF * CH, zmsg, 0, unroll=8)
    _zero_acc(msg_v, acc_sh, sid * NR)

    b1vec = b1_v[0, :]

    def mid(r, carry):
        d2 = plsc.load_gather(ds2_v, [jnp.full((16,), r, jnp.int32)])
        z = p1a_v[r, :] + p1b_v[r, :] + d2 * h1_v[r, :] + b1vec
        z1 = jnp.maximum(z, 0.0)
        z1_v[r, :] = z1
        u_v[r, :] = d2 * z1
        return carry

    lax.fori_loop(0, NBL, mid, 0, unroll=4)

    @pl.when(sid < 15)
    def _():
        nr = pl.ds(sid * NB, NB)
        vr = pl.ds(0, NB)
        pltpu.sync_copy(z1_v.at[vr], z1_sh.at[nr])

    @pl.when(sid == 15)
    def _():
        pltpu.sync_copy(z1_v, z1_sh.at[pl.ds(15 * NB, NBL)])

    @pl.when(jnp.logical_and(cid == 0, sid < 15))
    def _():
        nr = pl.ds(sid * NB, NB)
        pltpu.sync_copy(u_v.at[pl.ds(0, NB)], u_hbm.at[nr])

    @pl.when(jnp.logical_and(cid == 0, sid == 15))
    def _():
        pltpu.sync_copy(u_v, u_hbm.at[pl.ds(15 * NB, NBL)])

    def unpack(i, carry):
        c = i // (CH // 16)
        g = i % (CH // 16)
        p16 = pc_v[pl.ds(i * 16, 16)]
        row_v[pl.ds(i * 16, 16)] = p16 >> 14
        col2_v[c, pl.ds(g * 16, 16)] = p16 & PMASK
        return carry

    lax.fori_loop(0, NCHUNK * (CH // 16), unpack, 0, unroll=8)
    plsc.subcore_barrier()

    _edge_pass(z1_sh, acc_sh, row_v, col2_v, cw_v, msg_v,
               gsems, ssems)
    plsc.subcore_barrier()
    pltpu.sync_copy(acc_sh.at[pl.ds(sid * NR, NR)],
                    p2_hbm.at[cid, pl.ds(sid * NR, NR)])


_BR = 2000


def _tc_mm_body(x_ref, w1_ref, ei_ref, h1_ref, pc_ref):
    h1_ref[...] = jnp.dot(x_ref[...], w1_ref[...],
                          preferred_element_type=jnp.float32)

    @pl.when(pl.program_id(0) == 0)
    def _():
        pc_ref[...] = (ei_ref[0] << 14) | ei_ref[1]


def _tc_mm(x, w1, ei):
    return pl.pallas_call(
        _tc_mm_body,
        grid=(NN // _BR,),
        in_specs=[
            pl.BlockSpec((_BR, DF), lambda i: (i, 0)),
            pl.BlockSpec((DF, DH), lambda i: (0, 0)),
            pl.BlockSpec((2, NE), lambda i: (0, 0)),
        ],
        out_specs=[
            pl.BlockSpec((_BR, DH), lambda i: (i, 0)),
            pl.BlockSpec((NE,), lambda i: (0,)),
        ],
        out_shape=[
            jax.ShapeDtypeStruct((NN, DH), jnp.float32),
            jax.ShapeDtypeStruct((NE,), jnp.int32),
        ],
    )(x, w1, ei)


def _tc_post_body(p_ref, u_ref, w2_ref, b2_ref, out_ref):
    agg = p_ref[0] + p_ref[1] + u_ref[...]
    o = jnp.dot(agg, w2_ref[...], preferred_element_type=jnp.float32)
    o = o + b2_ref[...]
    m = jnp.max(o, axis=1, keepdims=True)
    lse = jnp.log(jnp.sum(jnp.exp(o - m), axis=1, keepdims=True)) + m
    out_ref[...] = o - lse


def _tc_post(p, u, w2, b2):
    return pl.pallas_call(
        _tc_post_body,
        grid=(NN // _BR,),
        in_specs=[
            pl.BlockSpec((NCORES, _BR, DH), lambda i: (0, i, 0)),
            pl.BlockSpec((_BR, DH), lambda i: (i, 0)),
            pl.BlockSpec((DH, DC), lambda i: (0, 0)),
            pl.BlockSpec((1, DC), lambda i: (0, 0)),
        ],
        out_specs=pl.BlockSpec((_BR, DC), lambda i: (i, 0)),
        out_shape=jax.ShapeDtypeStruct((NN, DC), jnp.float32),
    )(p, u, w2, b2)


def kernel(x, edge_index, edge_attr, W1, b1, W2, b2):
    ei = edge_index.astype(jnp.int32)
    ea = edge_attr.astype(jnp.float32)

    h1, pc = _tc_mm(x, W1, ei)
    p1, cw, ds2 = _sc_k1(h1, pc, ea)
    p2, u = _sc_k2(p1, h1, ds2, b1.reshape(1, DH), pc, cw)
    return _tc_post(p2, u, W2, b2.reshape(1, DC))


# speedup vs baseline: 1.1354x; 1.0459x over previous
import functools

import jax
import jax.numpy as jnp
from jax import lax
from jax.experimental import pallas as pl
from jax.experimental.pallas import tpu as pltpu
from jax.experimental.pallas import tpu_sc as plsc

NN = 10000
NE = 320000
DF = 128
DH = 16
DC = 40

NCORES = 2
NSUB = 16
NW = NCORES * NSUB
EPT = NE // NW
EPC = NE // NSUB
CH = 80
NCHUNK = EPT // CH
NBUF = 5
DROW = NN // 16
DPAD = 640
NR = NN // NSUB
NB = 624
NBL = NN - 15 * NB

_mesh = plsc.VectorSubcoreMesh(core_axis_name="c", subcore_axis_name="s")
_sc_params = pltpu.CompilerParams(needs_layout_passes=False,
                                  use_tc_tiling_on_sc=False)


def _newton_rsqrt(x):
    i = plsc.bitcast(x, jnp.int32)
    i = jnp.int32(0x5F3759DF) - (i >> 1)
    y = plsc.bitcast(i, jnp.float32)
    xh = x * 0.5
    for _ in range(4):
        y = y * (1.5 - xh * y * y)
    return y


def _zero_acc(msg_v, acc_sh, base):
    for j in range(7):
        pltpu.sync_copy(msg_v.at[0], acc_sh.at[pl.ds(base + j * CH, CH)])
    pltpu.sync_copy(msg_v.at[1].at[pl.ds(0, NR - 7 * CH)],
                    acc_sh.at[pl.ds(base + 7 * CH, NR - 7 * CH)])


def _edge_pass(h_src, acc_sh, row_v, col2_v, cw_v, msg_v, gsems, ssems):
    def start_gather(c, b):
        pltpu.async_copy(h_src.at[row_v.at[pl.ds(c * CH, CH)]],
                         msg_v.at[b], gsems[b])

    def wait_gather(c, b):
        pltpu.make_async_copy(h_src.at[row_v.at[pl.ds(c * CH, CH)]],
                              msg_v.at[b], gsems[b]).wait()

    def start_scatter(c, b):
        pltpu.async_copy(msg_v.at[b], acc_sh.at[col2_v.at[c]], ssems[b],
                         add=True)

    def wait_scatter(c, b):
        pltpu.make_async_copy(msg_v.at[b], acc_sh.at[col2_v.at[c]],
                              ssems[b]).wait()

    for b in range(3):
        start_gather(b, b)

    def group(g, carry):
        for b in range(NBUF):
            c = g * NBUF + b
            bn = (b + 3) % NBUF
            if b < 2:
                @pl.when(g >= 1)
                def _():
                    wait_scatter(c - 2, bn)
                start_gather(c + 3, bn)
            else:
                @pl.when(g < (NCHUNK // NBUF) - 1)
                def _():
                    wait_scatter(c - 2, bn)
                    start_gather(c + 3, bn)
            wait_gather(c, b)
            for gg in range(CH // 16):
                wvec = cw_v[pl.ds(c * CH + gg * 16, 16)]
                for e in range(16):
                    r = gg * 16 + e
                    msg_v[b, r, :] = msg_v[b, r, :] * wvec[e]
            start_scatter(c, b)
        return carry

    lax.fori_loop(0, NCHUNK // NBUF, group, 0)
    for c in range(NCHUNK - NBUF, NCHUNK):
        wait_scatter(c, c % NBUF)


PMASK = (1 << 14) - 1


@functools.partial(
    pl.kernel,
    out_type=[
        jax.ShapeDtypeStruct((NCORES, NN, DH), jnp.float32),
        jax.ShapeDtypeStruct((NE,), jnp.float32),
        jax.ShapeDtypeStruct((NN,), jnp.float32),
    ],
    mesh=_mesh,
    compiler_params=_sc_params,
    scratch_types=[
        pltpu.VMEM((EPC,), jnp.int32),
        pltpu.VMEM((EPC,), jnp.float32),
        pltpu.VMEM((DROW, 16), jnp.float32),
        pltpu.VMEM((5, 125), jnp.int32),
        pltpu.VMEM((NN,), jnp.float32),
        pltpu.VMEM((NN,), jnp.float32),
        pltpu.VMEM((EPT,), jnp.int32),
        pltpu.VMEM((NCHUNK, CH), jnp.int32),
        pltpu.VMEM((EPT,), jnp.float32),
        pltpu.VMEM((NBUF, CH, DH), jnp.float32),
        pltpu.VMEM_SHARED((DPAD, 16), jnp.float32),
        pltpu.VMEM_SHARED((NN, DH), jnp.float32),
        [pltpu.SemaphoreType.DMA] * NBUF,
        [pltpu.SemaphoreType.DMA] * NBUF,
    ],
)
def _sc_k1(h1_hbm, pc_hbm, ea_hbm, p1_hbm, cw_hbm, ds2_hbm,
           dpc_v, dew_v, dacc_v, idx2_v, disp_v, ds2_v,
           row_v, col2_v, cw_v, msg_v,
           deg_sh, acc_sh, gsems, ssems):
    cid = lax.axis_index("c")
    sid = lax.axis_index("s")
    wid = sid * NCORES + cid
    eoff = cid * EPT

    cp_pc = pltpu.async_copy(pc_hbm.at[pl.ds(sid * EPC, EPC)], dpc_v,
                             gsems[0])
    cp_ew = pltpu.async_copy(ea_hbm.at[pl.ds(sid * EPC, EPC)], dew_v,
                             gsems[1])

    zeros = jnp.zeros((16,), jnp.float32)

    def zmsg(i, carry):
        msg_v[i // CH, i % CH, :] = zeros
        return carry

    lax.fori_loop(0, NBUF * CH, zmsg, 0, unroll=8)

    def zacc(r, carry):
        dacc_v[r, :] = zeros
        return carry

    lax.fori_loop(0, DROW, zacc, 0, unroll=8)

    iota = lax.iota(jnp.int32, 16)
    for j in range(5):
        for i in range(8):
            base = min(i * 16, 125 - 16)
            idx2_v[j, pl.ds(base, 16)] = iota + (j * 125 + base)

    _zero_acc(msg_v, acc_sh, sid * NR)
    pltpu.sync_copy(msg_v.at[2].at[pl.ds(0, DPAD // NSUB)],
                    deg_sh.at[pl.ds(sid * (DPAD // NSUB), DPAD // NSUB)])
    cp_pc.wait()
    cp_ew.wait()
    plsc.subcore_barrier()

    def dbody(i, carry):
        c16 = dpc_v[pl.ds(i * 16, 16)] & PMASK
        w16 = dew_v[pl.ds(i * 16, 16)]
        plsc.addupdate_scatter(dacc_v, [c16 >> 4, c16 & 15], w16)
        return carry

    lax.fori_loop(0, EPC // 16, dbody, 0, unroll=4)
    for j in range(5):
        pltpu.sync_copy(dacc_v.at[pl.ds(j * 125, 125)],
                        deg_sh.at[idx2_v.at[j]], add=True)
    plsc.subcore_barrier()

    pltpu.sync_copy(deg_sh.at[pl.ds(0, DROW)], dacc_v)

    def newton(r, carry):
        y = _newton_rsqrt(dacc_v[r, :] + 1.0)
        disp_v[pl.ds(r * 16, 16)] = y
        ds2_v[pl.ds(r * 16, 16)] = y * y
        return carry

    lax.fori_loop(0, DROW, newton, 0, unroll=4)

    def cwbody(i, carry):
        s = pl.ds(i * 16, 16)
        p16 = dpc_v[pl.ds(eoff + i * 16, 16)]
        r16 = p16 >> 14
        c16 = p16 & PMASK
        row_v[s] = r16
        dr = plsc.load_gather(disp_v, [r16])
        dc = plsc.load_gather(disp_v, [c16])
        cw_v[s] = dew_v[pl.ds(eoff + i * 16, 16)] * dr * dc
        return carry

    lax.fori_loop(0, EPT // 16, cwbody, 0, unroll=4)
    pltpu.sync_copy(cw_v, cw_hbm.at[pl.ds(wid * EPT, EPT)])

    @pl.when(jnp.logical_and(cid == 0, sid < 15))
    def _():
        pltpu.sync_copy(ds2_v.at[pl.ds(sid * NB, NB)],
                        ds2_hbm.at[pl.ds(sid * NB, NB)])

    @pl.when(jnp.logical_and(cid == 0, sid == 15))
    def _():
        pltpu.sync_copy(ds2_v.at[pl.ds(15 * NB, NBL)],
                        ds2_hbm.at[pl.ds(15 * NB, NBL)])

    def repack(i, carry):
        c = i // (CH // 16)
        g = i % (CH // 16)
        col2_v[c, pl.ds(g * 16, 16)] = dpc_v[pl.ds(eoff + i * 16, 16)] & PMASK
        return carry

    lax.fori_loop(0, NCHUNK * (CH // 16), repack, 0, unroll=8)

    _edge_pass(h1_hbm, acc_sh, row_v, col2_v, cw_v, msg_v, gsems, ssems)
    plsc.subcore_barrier()
    pltpu.sync_copy(acc_sh.at[pl.ds(sid * NR, NR)],
                    p1_hbm.at[cid, pl.ds(sid * NR, NR)])


@functools.partial(
    pl.kernel,
    out_type=[
        jax.ShapeDtypeStruct((NCORES, NN, DH), jnp.float32),
        jax.ShapeDtypeStruct((NN, DH), jnp.float32),
    ],
    mesh=_mesh,
    compiler_params=_sc_params,
    scratch_types=[
        pltpu.VMEM((EPT,), jnp.int32),
        pltpu.VMEM((EPT,), jnp.int32),
        pltpu.VMEM((NCHUNK, CH), jnp.int32),
        pltpu.VMEM((EPT,), jnp.float32),
        pltpu.VMEM((NBUF, CH, DH), jnp.float32),
        pltpu.VMEM((NBL, DH), jnp.float32),
        pltpu.VMEM((NBL, DH), jnp.float32),
        pltpu.VMEM((NBL, DH), jnp.float32),
        pltpu.VMEM((NBL,), jnp.float32),
        pltpu.VMEM((NBL, DH), jnp.float32),
        pltpu.VMEM((NBL, DH), jnp.float32),
        pltpu.VMEM((1, DH), jnp.float32),
        pltpu.VMEM_SHARED((NN, DH), jnp.float32),
        pltpu.VMEM_SHARED((NN, DH), jnp.float32),
        [pltpu.SemaphoreType.DMA] * NBUF,
        [pltpu.SemaphoreType.DMA] * NBUF,
    ],
)
def _sc_k2(p1_hbm, h1_hbm, ds2_hbm, b1_hbm, pc_hbm, cw_hbm,
           p2_hbm, u_hbm,
           pc_v, row_v, col2_v, cw_v, msg_v,
           p1a_v, p1b_v, h1_v, ds2_v, z1_v, u_v, b1_v,
           z1_sh, acc_sh, gsems, ssems):
    cid = lax.axis_index("c")
    sid = lax.axis_index("s")
    wid = sid * NCORES + cid

    cp_pc = pltpu.async_copy(pc_hbm.at[pl.ds(wid * EPT, EPT)], pc_v,
                             gsems[0])
    cp_cw = pltpu.async_copy(cw_hbm.at[pl.ds(wid * EPT, EPT)], cw_v,
                             gsems[1])
    cp_b1 = pltpu.async_copy(b1_hbm, b1_v, gsems[2])

    @pl.when(sid < 15)
    def _():
        nr = pl.ds(sid * NB, NB)
        vr = pl.ds(0, NB)
        pltpu.async_copy(p1_hbm.at[0, nr], p1a_v.at[vr], gsems[3])
        pltpu.async_copy(p1_hbm.at[1, nr], p1b_v.at[vr], gsems[4])
        pltpu.async_copy(h1_hbm.at[nr], h1_v.at[vr], ssems[0])
        pltpu.async_copy(ds2_hbm.at[nr], ds2_v.at[vr], ssems[1])

    @pl.when(sid == 15)
    def _():
        nr = pl.ds(15 * NB, NBL)
        pltpu.async_copy(p1_hbm.at[0, nr], p1a_v, gsems[3])
        pltpu.async_copy(p1_hbm.at[1, nr], p1b_v, gsems[4])
        pltpu.async_copy(h1_hbm.at[nr], h1_v, ssems[0])
        pltpu.async_copy(ds2_hbm.at[nr], ds2_v, ssems[1])

    zeros = jnp.zeros((16,), jnp.float32)

    def zmsg(i, carry):
        msg_v[i // CH, i % CH, :] = zeros
        return carry

    lax.fori_loop(0, NBUF * CH, zmsg, 0, unroll=8)
    _zero_acc(msg_v, acc_sh, sid * NR)

    @pl.when(sid < 15)
    def _():
        nr = pl.ds(sid * NB, NB)
        vr = pl.ds(0, NB)
        pltpu.make_async_copy(p1_hbm.at[0, nr], p1a_v.at[vr], gsems[3]).wait()
        pltpu.make_async_copy(p1_hbm.at[1, nr], p1b_v.at[vr], gsems[4]).wait()
        pltpu.make_async_copy(h1_hbm.at[nr], h1_v.at[vr], ssems[0]).wait()
        pltpu.make_async_copy(ds2_hbm.at[nr], ds2_v.at[vr], ssems[1]).wait()

    @pl.when(sid == 15)
    def _():
        nr = pl.ds(15 * NB, NBL)
        pltpu.make_async_copy(p1_hbm.at[0, nr], p1a_v, gsems[3]).wait()
        pltpu.make_async_copy(p1_hbm.at[1, nr], p1b_v, gsems[4]).wait()
        pltpu.make_async_copy(h1_hbm.at[nr], h1_v, ssems[0]).wait()
        pltpu.make_async_copy(ds2_hbm.at[nr], ds2_v, ssems[1]).wait()

    cp_b1.wait()
    b1vec = b1_v[0, :]

    def mid(r, carry):
        d2 = plsc.load_gather(ds2_v, [jnp.full((16,), r, jnp.int32)])
        z = p1a_v[r, :] + p1b_v[r, :] + d2 * h1_v[r, :] + b1vec
        z1 = jnp.maximum(z, 0.0)
        z1_v[r, :] = z1
        u_v[r, :] = d2 * z1
        return carry

    lax.fori_loop(0, NBL, mid, 0, unroll=4)

    @pl.when(sid < 15)
    def _():
        nr = pl.ds(sid * NB, NB)
        vr = pl.ds(0, NB)
        pltpu.sync_copy(z1_v.at[vr], z1_sh.at[nr])

    @pl.when(sid == 15)
    def _():
        pltpu.sync_copy(z1_v, z1_sh.at[pl.ds(15 * NB, NBL)])

    @pl.when(jnp.logical_and(cid == 0, sid < 15))
    def _():
        nr = pl.ds(sid * NB, NB)
        pltpu.sync_copy(u_v.at[pl.ds(0, NB)], u_hbm.at[nr])

    @pl.when(jnp.logical_and(cid == 0, sid == 15))
    def _():
        pltpu.sync_copy(u_v, u_hbm.at[pl.ds(15 * NB, NBL)])

    cp_pc.wait()

    def unpack(i, carry):
        c = i // (CH // 16)
        g = i % (CH // 16)
        p16 = pc_v[pl.ds(i * 16, 16)]
        row_v[pl.ds(i * 16, 16)] = p16 >> 14
        col2_v[c, pl.ds(g * 16, 16)] = p16 & PMASK
        return carry

    lax.fori_loop(0, NCHUNK * (CH // 16), unpack, 0, unroll=8)
    cp_cw.wait()
    plsc.subcore_barrier()

    _edge_pass(z1_sh, acc_sh, row_v, col2_v, cw_v, msg_v,
               gsems, ssems)
    plsc.subcore_barrier()
    pltpu.sync_copy(acc_sh.at[pl.ds(sid * NR, NR)],
                    p2_hbm.at[cid, pl.ds(sid * NR, NR)])


_BR = 2000


def _tc_mm_body(x_ref, w1_ref, ei_ref, h1_ref, pc_ref):
    h1_ref[...] = jnp.dot(x_ref[...], w1_ref[...],
                          preferred_element_type=jnp.float32)

    @pl.when(pl.program_id(0) == 0)
    def _():
        pc_ref[...] = (ei_ref[0] << 14) | ei_ref[1]


def _tc_mm(x, w1, ei):
    return pl.pallas_call(
        _tc_mm_body,
        grid=(NN // _BR,),
        in_specs=[
            pl.BlockSpec((_BR, DF), lambda i: (i, 0)),
            pl.BlockSpec((DF, DH), lambda i: (0, 0)),
            pl.BlockSpec((2, NE), lambda i: (0, 0)),
        ],
        out_specs=[
            pl.BlockSpec((_BR, DH), lambda i: (i, 0)),
            pl.BlockSpec((NE,), lambda i: (0,)),
        ],
        out_shape=[
            jax.ShapeDtypeStruct((NN, DH), jnp.float32),
            jax.ShapeDtypeStruct((NE,), jnp.int32),
        ],
    )(x, w1, ei)


def _tc_post_body(p_ref, u_ref, w2_ref, b2_ref, out_ref):
    agg = p_ref[0] + p_ref[1] + u_ref[...]
    o = jnp.dot(agg, w2_ref[...], preferred_element_type=jnp.float32)
    o = o + b2_ref[...]
    m = jnp.max(o, axis=1, keepdims=True)
    lse = jnp.log(jnp.sum(jnp.exp(o - m), axis=1, keepdims=True)) + m
    out_ref[...] = o - lse


def _tc_post(p, u, w2, b2):
    return pl.pallas_call(
        _tc_post_body,
        grid=(NN // _BR,),
        in_specs=[
            pl.BlockSpec((NCORES, _BR, DH), lambda i: (0, i, 0)),
            pl.BlockSpec((_BR, DH), lambda i: (i, 0)),
            pl.BlockSpec((DH, DC), lambda i: (0, 0)),
            pl.BlockSpec((1, DC), lambda i: (0, 0)),
        ],
        out_specs=pl.BlockSpec((_BR, DC), lambda i: (i, 0)),
        out_shape=jax.ShapeDtypeStruct((NN, DC), jnp.float32),
    )(p, u, w2, b2)


def kernel(x, edge_index, edge_attr, W1, b1, W2, b2):
    ei = edge_index.astype(jnp.int32)
    ea = edge_attr.astype(jnp.float32)

    h1, pc = _tc_mm(x, W1, ei)
    p1, cw, ds2 = _sc_k1(h1, pc, ea)
    p2, u = _sc_k2(p1, h1, ds2, b1.reshape(1, DH), pc, cw)
    return _tc_post(p2, u, W2, b2.reshape(1, DC))


# speedup vs baseline: 1.1615x; 1.0230x over previous
import functools

import jax
import jax.numpy as jnp
from jax import lax
from jax.experimental import pallas as pl
from jax.experimental.pallas import tpu as pltpu
from jax.experimental.pallas import tpu_sc as plsc

NN = 10000
NE = 320000
DF = 128
DH = 16
DC = 40

NCORES = 2
NSUB = 16
NW = NCORES * NSUB
EPT = NE // NW
EPC = NE // NSUB
CH = 80
NCHUNK = EPT // CH
NBUF = 5
DROW = NN // 16
DPAD = 640
NR = NN // NSUB
NB = 624
NBL = NN - 15 * NB

_mesh = plsc.VectorSubcoreMesh(core_axis_name="c", subcore_axis_name="s")
_sc_params = pltpu.CompilerParams(needs_layout_passes=False,
                                  use_tc_tiling_on_sc=False)


def _newton_rsqrt(x):
    i = plsc.bitcast(x, jnp.int32)
    i = jnp.int32(0x5F3759DF) - (i >> 1)
    y = plsc.bitcast(i, jnp.float32)
    xh = x * 0.5
    for _ in range(4):
        y = y * (1.5 - xh * y * y)
    return y


def _zero_acc(msg_v, acc_sh, base):
    for j in range(7):
        pltpu.sync_copy(msg_v.at[0], acc_sh.at[pl.ds(base + j * CH, CH)])
    pltpu.sync_copy(msg_v.at[1].at[pl.ds(0, NR - 7 * CH)],
                    acc_sh.at[pl.ds(base + 7 * CH, NR - 7 * CH)])


def _edge_pass(h_src, acc_sh, row_v, col2_v, cw_v, msg_v, gsems, ssems):
    def start_gather(c, b):
        pltpu.async_copy(h_src.at[row_v.at[pl.ds(c * CH, CH)]],
                         msg_v.at[b], gsems[b])

    def wait_gather(c, b):
        pltpu.make_async_copy(h_src.at[row_v.at[pl.ds(c * CH, CH)]],
                              msg_v.at[b], gsems[b]).wait()

    def start_scatter(c, b):
        pltpu.async_copy(msg_v.at[b], acc_sh.at[col2_v.at[c]], ssems[b],
                         add=True)

    def wait_scatter(c, b):
        pltpu.make_async_copy(msg_v.at[b], acc_sh.at[col2_v.at[c]],
                              ssems[b]).wait()

    for b in range(3):
        start_gather(b, b)

    def group(g, carry):
        for b in range(NBUF):
            c = g * NBUF + b
            bn = (b + 3) % NBUF
            if b < 2:
                @pl.when(g >= 1)
                def _():
                    wait_scatter(c - 2, bn)
                start_gather(c + 3, bn)
            else:
                @pl.when(g < (NCHUNK // NBUF) - 1)
                def _():
                    wait_scatter(c - 2, bn)
                    start_gather(c + 3, bn)
            wait_gather(c, b)
            for gg in range(CH // 16):
                wvec = cw_v[pl.ds(c * CH + gg * 16, 16)]
                for e in range(16):
                    r = gg * 16 + e
                    msg_v[b, r, :] = msg_v[b, r, :] * wvec[e]
            start_scatter(c, b)
        return carry

    lax.fori_loop(0, NCHUNK // NBUF, group, 0)
    for c in range(NCHUNK - NBUF, NCHUNK):
        wait_scatter(c, c % NBUF)


PMASK = (1 << 14) - 1


@functools.partial(
    pl.kernel,
    out_type=[
        jax.ShapeDtypeStruct((NE,), jnp.float32),
        jax.ShapeDtypeStruct((NN,), jnp.float32),
    ],
    mesh=_mesh,
    compiler_params=_sc_params,
    scratch_types=[
        pltpu.VMEM((EPC,), jnp.int32),
        pltpu.VMEM((EPC,), jnp.float32),
        pltpu.VMEM((EPT,), jnp.int32),
        pltpu.VMEM((DROW, 16), jnp.float32),
        pltpu.VMEM((5, 125), jnp.int32),
        pltpu.VMEM((40, 16), jnp.float32),
        pltpu.VMEM((NN,), jnp.float32),
        pltpu.VMEM((NN,), jnp.float32),
        pltpu.VMEM((EPT,), jnp.float32),
        pltpu.VMEM_SHARED((DPAD, 16), jnp.float32),
        [pltpu.SemaphoreType.DMA] * 3,
    ],
)
def _sc_deg(ei_hbm, ea_hbm, cw_hbm, ds2_hbm,
            dcol_v, dew_v, drow_v, dacc_v, idx2_v, zero_v, disp_v, ds2_v,
            cw_v, deg_sh, sems):
    cid = lax.axis_index("c")
    sid = lax.axis_index("s")
    wid = sid * NCORES + cid
    eoff = cid * EPT

    cp_c = pltpu.async_copy(ei_hbm.at[1, pl.ds(sid * EPC, EPC)], dcol_v,
                            sems[0])
    cp_w = pltpu.async_copy(ea_hbm.at[pl.ds(sid * EPC, EPC)], dew_v,
                            sems[1])
    cp_r = pltpu.async_copy(ei_hbm.at[0, pl.ds(wid * EPT, EPT)], drow_v,
                            sems[2])

    zeros = jnp.zeros((16,), jnp.float32)

    def zacc(r, carry):
        dacc_v[r, :] = zeros
        return carry

    lax.fori_loop(0, DROW, zacc, 0, unroll=8)
    for r in range(40):
        zero_v[r, :] = zeros

    iota = lax.iota(jnp.int32, 16)
    for j in range(5):
        for i in range(8):
            base = min(i * 16, 125 - 16)
            idx2_v[j, pl.ds(base, 16)] = iota + (j * 125 + base)

    pltpu.sync_copy(zero_v,
                    deg_sh.at[pl.ds(sid * (DPAD // NSUB), DPAD // NSUB)])
    cp_c.wait()
    cp_w.wait()
    plsc.subcore_barrier()

    def dbody(i, carry):
        c16 = dcol_v[pl.ds(i * 16, 16)]
        w16 = dew_v[pl.ds(i * 16, 16)]
        plsc.addupdate_scatter(dacc_v, [c16 >> 4, c16 & 15], w16)
        return carry

    lax.fori_loop(0, EPC // 16, dbody, 0, unroll=4)
    for j in range(5):
        pltpu.sync_copy(dacc_v.at[pl.ds(j * 125, 125)],
                        deg_sh.at[idx2_v.at[j]], add=True)
    plsc.subcore_barrier()

    pltpu.sync_copy(deg_sh.at[pl.ds(0, DROW)], dacc_v)

    def newton(r, carry):
        y = _newton_rsqrt(dacc_v[r, :] + 1.0)
        disp_v[pl.ds(r * 16, 16)] = y
        ds2_v[pl.ds(r * 16, 16)] = y * y
        return carry

    lax.fori_loop(0, DROW, newton, 0, unroll=4)
    cp_r.wait()

    def cwbody(i, carry):
        s = pl.ds(i * 16, 16)
        r16 = drow_v[s]
        c16 = dcol_v[pl.ds(eoff + i * 16, 16)]
        dr = plsc.load_gather(disp_v, [r16])
        dc = plsc.load_gather(disp_v, [c16])
        cw_v[s] = dew_v[pl.ds(eoff + i * 16, 16)] * dr * dc
        return carry

    lax.fori_loop(0, EPT // 16, cwbody, 0, unroll=4)
    pltpu.sync_copy(cw_v, cw_hbm.at[pl.ds(wid * EPT, EPT)])

    @pl.when(jnp.logical_and(cid == 0, sid < 15))
    def _():
        pltpu.sync_copy(ds2_v.at[pl.ds(sid * NB, NB)],
                        ds2_hbm.at[pl.ds(sid * NB, NB)])

    @pl.when(jnp.logical_and(cid == 0, sid == 15))
    def _():
        pltpu.sync_copy(ds2_v.at[pl.ds(15 * NB, NBL)],
                        ds2_hbm.at[pl.ds(15 * NB, NBL)])


@functools.partial(
    pl.kernel,
    out_type=jax.ShapeDtypeStruct((NCORES, NN, DH), jnp.float32),
    mesh=_mesh,
    compiler_params=_sc_params,
    scratch_types=[
        pltpu.VMEM((EPT,), jnp.int32),
        pltpu.VMEM((EPT,), jnp.int32),
        pltpu.VMEM((NCHUNK, CH), jnp.int32),
        pltpu.VMEM((EPT,), jnp.float32),
        pltpu.VMEM((NBUF, CH, DH), jnp.float32),
        pltpu.VMEM_SHARED((NN, DH), jnp.float32),
        [pltpu.SemaphoreType.DMA] * NBUF,
        [pltpu.SemaphoreType.DMA] * NBUF,
    ],
)
def _sc_k1(h1_hbm, pc_hbm, cw_hbm, p1_hbm,
           pc_v, row_v, col2_v, cw_v, msg_v,
           acc_sh, gsems, ssems):
    cid = lax.axis_index("c")
    sid = lax.axis_index("s")
    wid = sid * NCORES + cid

    cp_pc = pltpu.async_copy(pc_hbm.at[pl.ds(wid * EPT, EPT)], pc_v,
                             gsems[0])
    cp_cw = pltpu.async_copy(cw_hbm.at[pl.ds(wid * EPT, EPT)], cw_v,
                             gsems[1])

    zeros = jnp.zeros((16,), jnp.float32)

    def zmsg(i, carry):
        msg_v[i // CH, i % CH, :] = zeros
        return carry

    lax.fori_loop(0, NBUF * CH, zmsg, 0, unroll=8)
    _zero_acc(msg_v, acc_sh, sid * NR)
    cp_pc.wait()

    def repack(i, carry):
        c = i // (CH // 16)
        g = i % (CH // 16)
        p16 = pc_v[pl.ds(i * 16, 16)]
        row_v[pl.ds(i * 16, 16)] = p16 >> 14
        col2_v[c, pl.ds(g * 16, 16)] = p16 & PMASK
        return carry

    lax.fori_loop(0, NCHUNK * (CH // 16), repack, 0, unroll=8)
    cp_cw.wait()
    plsc.subcore_barrier()

    _edge_pass(h1_hbm, acc_sh, row_v, col2_v, cw_v, msg_v, gsems, ssems)
    plsc.subcore_barrier()
    pltpu.sync_copy(acc_sh.at[pl.ds(sid * NR, NR)],
                    p1_hbm.at[cid, pl.ds(sid * NR, NR)])


@functools.partial(
    pl.kernel,
    out_type=[
        jax.ShapeDtypeStruct((NCORES, NN, DH), jnp.float32),
        jax.ShapeDtypeStruct((NN, DH), jnp.float32),
    ],
    mesh=_mesh,
    compiler_params=_sc_params,
    scratch_types=[
        pltpu.VMEM((EPT,), jnp.int32),
        pltpu.VMEM((EPT,), jnp.int32),
        pltpu.VMEM((NCHUNK, CH), jnp.int32),
        pltpu.VMEM((EPT,), jnp.float32),
        pltpu.VMEM((NBUF, CH, DH), jnp.float32),
        pltpu.VMEM((NBL, DH), jnp.float32),
        pltpu.VMEM((NBL, DH), jnp.float32),
        pltpu.VMEM((NBL, DH), jnp.float32),
        pltpu.VMEM((NBL,), jnp.float32),
        pltpu.VMEM((NBL, DH), jnp.float32),
        pltpu.VMEM((NBL, DH), jnp.float32),
        pltpu.VMEM((1, DH), jnp.float32),
        pltpu.VMEM_SHARED((NN, DH), jnp.float32),
        pltpu.VMEM_SHARED((NN, DH), jnp.float32),
        [pltpu.SemaphoreType.DMA] * NBUF,
        [pltpu.SemaphoreType.DMA] * NBUF,
    ],
)
def _sc_k2(p1_hbm, h1_hbm, ds2_hbm, b1_hbm, pc_hbm, cw_hbm,
           p2_hbm, u_hbm,
           pc_v, row_v, col2_v, cw_v, msg_v,
           p1a_v, p1b_v, h1_v, ds2_v, z1_v, u_v, b1_v,
           z1_sh, acc_sh, gsems, ssems):
    cid = lax.axis_index("c")
    sid = lax.axis_index("s")
    wid = sid * NCORES + cid

    cp_pc = pltpu.async_copy(pc_hbm.at[pl.ds(wid * EPT, EPT)], pc_v,
                             gsems[0])
    cp_cw = pltpu.async_copy(cw_hbm.at[pl.ds(wid * EPT, EPT)], cw_v,
                             gsems[1])
    cp_b1 = pltpu.async_copy(b1_hbm, b1_v, gsems[2])

    @pl.when(sid < 15)
    def _():
        nr = pl.ds(sid * NB, NB)
        vr = pl.ds(0, NB)
        pltpu.async_copy(p1_hbm.at[0, nr], p1a_v.at[vr], gsems[3])
        pltpu.async_copy(p1_hbm.at[1, nr], p1b_v.at[vr], gsems[4])
        pltpu.async_copy(h1_hbm.at[nr], h1_v.at[vr], ssems[0])
        pltpu.async_copy(ds2_hbm.at[nr], ds2_v.at[vr], ssems[1])

    @pl.when(sid == 15)
    def _():
        nr = pl.ds(15 * NB, NBL)
        pltpu.async_copy(p1_hbm.at[0, nr], p1a_v, gsems[3])
        pltpu.async_copy(p1_hbm.at[1, nr], p1b_v, gsems[4])
        pltpu.async_copy(h1_hbm.at[nr], h1_v, ssems[0])
        pltpu.async_copy(ds2_hbm.at[nr], ds2_v, ssems[1])

    zeros = jnp.zeros((16,), jnp.float32)

    def zmsg(i, carry):
        msg_v[i // CH, i % CH, :] = zeros
        return carry

    lax.fori_loop(0, NBUF * CH, zmsg, 0, unroll=8)
    _zero_acc(msg_v, acc_sh, sid * NR)

    @pl.when(sid < 15)
    def _():
        nr = pl.ds(sid * NB, NB)
        vr = pl.ds(0, NB)
        pltpu.make_async_copy(p1_hbm.at[0, nr], p1a_v.at[vr], gsems[3]).wait()
        pltpu.make_async_copy(p1_hbm.at[1, nr], p1b_v.at[vr], gsems[4]).wait()
        pltpu.make_async_copy(h1_hbm.at[nr], h1_v.at[vr], ssems[0]).wait()
        pltpu.make_async_copy(ds2_hbm.at[nr], ds2_v.at[vr], ssems[1]).wait()

    @pl.when(sid == 15)
    def _():
        nr = pl.ds(15 * NB, NBL)
        pltpu.make_async_copy(p1_hbm.at[0, nr], p1a_v, gsems[3]).wait()
        pltpu.make_async_copy(p1_hbm.at[1, nr], p1b_v, gsems[4]).wait()
        pltpu.make_async_copy(h1_hbm.at[nr], h1_v, ssems[0]).wait()
        pltpu.make_async_copy(ds2_hbm.at[nr], ds2_v, ssems[1]).wait()

    cp_b1.wait()
    b1vec = b1_v[0, :]

    def mid(r, carry):
        d2 = plsc.load_gather(ds2_v, [jnp.full((16,), r, jnp.int32)])
        z = p1a_v[r, :] + p1b_v[r, :] + d2 * h1_v[r, :] + b1vec
        z1 = jnp.maximum(z, 0.0)
        z1_v[r, :] = z1
        u_v[r, :] = d2 * z1
        return carry

    lax.fori_loop(0, NBL, mid, 0, unroll=4)

    @pl.when(sid < 15)
    def _():
        nr = pl.ds(sid * NB, NB)
        vr = pl.ds(0, NB)
        pltpu.sync_copy(z1_v.at[vr], z1_sh.at[nr])

    @pl.when(sid == 15)
    def _():
        pltpu.sync_copy(z1_v, z1_sh.at[pl.ds(15 * NB, NBL)])

    @pl.when(jnp.logical_and(cid == 0, sid < 15))
    def _():
        nr = pl.ds(sid * NB, NB)
        pltpu.sync_copy(u_v.at[pl.ds(0, NB)], u_hbm.at[nr])

    @pl.when(jnp.logical_and(cid == 0, sid == 15))
    def _():
        pltpu.sync_copy(u_v, u_hbm.at[pl.ds(15 * NB, NBL)])

    cp_pc.wait()

    def unpack(i, carry):
        c = i // (CH // 16)
        g = i % (CH // 16)
        p16 = pc_v[pl.ds(i * 16, 16)]
        row_v[pl.ds(i * 16, 16)] = p16 >> 14
        col2_v[c, pl.ds(g * 16, 16)] = p16 & PMASK
        return carry

    lax.fori_loop(0, NCHUNK * (CH // 16), unpack, 0, unroll=8)
    cp_cw.wait()
    plsc.subcore_barrier()

    _edge_pass(z1_sh, acc_sh, row_v, col2_v, cw_v, msg_v,
               gsems, ssems)
    plsc.subcore_barrier()
    pltpu.sync_copy(acc_sh.at[pl.ds(sid * NR, NR)],
                    p2_hbm.at[cid, pl.ds(sid * NR, NR)])


_BR = 2000


def _tc_mm_body(x_ref, w1_ref, ei_ref, h1_ref, pc_ref):
    h1_ref[...] = jnp.dot(x_ref[...], w1_ref[...],
                          preferred_element_type=jnp.float32)

    @pl.when(pl.program_id(0) == 0)
    def _():
        pc_ref[...] = (ei_ref[0] << 14) | ei_ref[1]


def _tc_mm(x, w1, ei):
    return pl.pallas_call(
        _tc_mm_body,
        grid=(NN // _BR,),
        in_specs=[
            pl.BlockSpec((_BR, DF), lambda i: (i, 0)),
            pl.BlockSpec((DF, DH), lambda i: (0, 0)),
            pl.BlockSpec((2, NE), lambda i: (0, 0)),
        ],
        out_specs=[
            pl.BlockSpec((_BR, DH), lambda i: (i, 0)),
            pl.BlockSpec((NE,), lambda i: (0,)),
        ],
        out_shape=[
            jax.ShapeDtypeStruct((NN, DH), jnp.float32),
            jax.ShapeDtypeStruct((NE,), jnp.int32),
        ],
    )(x, w1, ei)


def _tc_post_body(p_ref, u_ref, w2_ref, b2_ref, out_ref):
    agg = p_ref[0] + p_ref[1] + u_ref[...]
    o = jnp.dot(agg, w2_ref[...], preferred_element_type=jnp.float32)
    o = o + b2_ref[...]
    m = jnp.max(o, axis=1, keepdims=True)
    lse = jnp.log(jnp.sum(jnp.exp(o - m), axis=1, keepdims=True)) + m
    out_ref[...] = o - lse


def _tc_post(p, u, w2, b2):
    return pl.pallas_call(
        _tc_post_body,
        grid=(NN // _BR,),
        in_specs=[
            pl.BlockSpec((NCORES, _BR, DH), lambda i: (0, i, 0)),
            pl.BlockSpec((_BR, DH), lambda i: (i, 0)),
            pl.BlockSpec((DH, DC), lambda i: (0, 0)),
            pl.BlockSpec((1, DC), lambda i: (0, 0)),
        ],
        out_specs=pl.BlockSpec((_BR, DC), lambda i: (i, 0)),
        out_shape=jax.ShapeDtypeStruct((NN, DC), jnp.float32),
    )(p, u, w2, b2)


def kernel(x, edge_index, edge_attr, W1, b1, W2, b2):
    ei = edge_index.astype(jnp.int32)
    ea = edge_attr.astype(jnp.float32)

    cw, ds2 = _sc_deg(ei, ea)
    h1, pc = _tc_mm(x, W1, ei)
    p1 = _sc_k1(h1, pc, cw)
    p2, u = _sc_k2(p1, h1, ds2, b1.reshape(1, DH), pc, cw)
    return _tc_post(p2, u, W2, b2.reshape(1, DC))
